# baseline (speedup 1.0000x reference)
import functools

import jax
import jax.numpy as jnp
from jax import lax
from jax.experimental import pallas as pl
from jax.experimental.pallas import tpu as pltpu
from jax.experimental.pallas import tpu_sc as plsc

IMAGE_SIZE = 300
D_POS = 64
DG = 128
T = 100
B = 1024
ROWS = B * T
BBOX_DIM = 4 * D_POS
OUT_DIM = 512
ZROW = IMAGE_SIZE + 1

CHUNK_R = 128
NW = 32
R_PER_W = ROWS // NW
NCHUNK = R_PER_W // CHUNK_R
UNITS = NW * NCHUNK


def _sc_gather_body(pk_hbm, table_hbm, poe_hbm, pk_v, idx_v, rows_v, sem):
    wid = lax.axis_index("s") * 2 + lax.axis_index("c")
    r0 = wid * R_PER_W

    def chunk(ci, carry):
        off = r0 + ci * CHUNK_R
        pltpu.sync_copy(pk_hbm.at[:, pl.ds(off, CHUNK_R)], pk_v)
        for c in range(4):
            for v in range(CHUNK_R // 16):
                f = lax.bitcast_convert_type(
                    pk_v[c, pl.ds(v * 16, 16)], jnp.float32
                )
                o = pk_v[4, pl.ds(v * 16, 16)]
                xi = (f * float(IMAGE_SIZE)).astype(jnp.int32)
                xi = jnp.minimum(jnp.maximum(xi, 0), IMAGE_SIZE)
                xi = jnp.where(o == 1, xi, ZROW)
                idx_v[c, pl.ds(v * 16, 16)] = xi
        descs = [
            pltpu.async_copy(table_hbm.at[idx_v.at[c]], rows_v.at[c], sem)
            for c in range(4)
        ]
        for d in descs:
            d.wait()
        pltpu.sync_copy(rows_v, poe_hbm.at[wid * NCHUNK + ci])
        return carry

    lax.fori_loop(0, NCHUNK, chunk, 0)


def _sc_gather(packed, table):
    mesh = plsc.VectorSubcoreMesh(core_axis_name="c", subcore_axis_name="s")
    return functools.partial(
        pl.kernel,
        mesh=mesh,
        compiler_params=pltpu.CompilerParams(use_tc_tiling_on_sc=False),
        out_type=jax.ShapeDtypeStruct((UNITS, 4, CHUNK_R, DG), jnp.float32),
        scratch_types=[
            pltpu.VMEM((5, CHUNK_R), jnp.int32),
            pltpu.VMEM((4, CHUNK_R), jnp.int32),
            pltpu.VMEM((4, CHUNK_R, DG), jnp.float32),
            pltpu.SemaphoreType.DMA,
        ],
    )(_sc_gather_body)(packed, table)


def _base_body(wt_ref, ww_ref, b_ref, out_ref):
    acc = (
        jnp.dot(wt_ref[...], ww_ref[...], preferred_element_type=jnp.float32)
        + b_ref[...]
    )
    for k in range(R_PER_W // T):
        out_ref[pl.ds(k * T, T), :] = acc


def _mm_body(poe_ref, wq_ref, base_ref, out_ref):
    acc = base_ref[0]
    for c in range(4):
        acc = acc + jnp.dot(
            poe_ref[0, c], wq_ref[c], preferred_element_type=jnp.float32
        )
    out_ref[...] = jnp.maximum(acc, 0.0)


def kernel(batch_fractional_bboxs, batch_obj_vecs, pos_table, word_table, W, b):
    frac_t = batch_fractional_bboxs.reshape(ROWS, 4).T
    packed = jnp.concatenate(
        [lax.bitcast_convert_type(frac_t, jnp.int32),
         batch_obj_vecs.reshape(1, ROWS)], axis=0)
    table = jnp.pad(pos_table, ((0, 3), (0, DG - D_POS)))
    Wq = jnp.pad(
        W[:BBOX_DIM].reshape(4, D_POS, OUT_DIM),
        ((0, 0), (0, DG - D_POS), (0, 0)),
    )
    Ww = W[BBOX_DIM:]

    base_per = pl.pallas_call(
        _base_body,
        out_shape=jax.ShapeDtypeStruct((R_PER_W, OUT_DIM), jnp.float32),
    )(word_table, Ww, b.reshape(1, OUT_DIM))
    base25 = base_per.reshape(NCHUNK, CHUNK_R, OUT_DIM)

    poe = _sc_gather(packed, table)

    return pl.pallas_call(
        _mm_body,
        grid=(NCHUNK, NW),
        in_specs=[
            pl.BlockSpec((1, 4, CHUNK_R, DG),
                         lambda ci, w: (w * NCHUNK + ci, 0, 0, 0)),
            pl.BlockSpec((4, DG, OUT_DIM), lambda ci, w: (0, 0, 0)),
            pl.BlockSpec((1, CHUNK_R, OUT_DIM), lambda ci, w: (ci, 0, 0)),
        ],
        out_specs=pl.BlockSpec((CHUNK_R, OUT_DIM),
                               lambda ci, w: (w * NCHUNK + ci, 0)),
        out_shape=jax.ShapeDtypeStruct((ROWS, OUT_DIM), jnp.float32),
    )(poe, Wq, base25)


# speedup vs baseline: 10.0188x; 10.0188x over previous
import functools

import jax
import jax.numpy as jnp
from jax import lax
from jax.experimental import pallas as pl
from jax.experimental.pallas import tpu as pltpu
from jax.experimental.pallas import tpu_sc as plsc

IMAGE_SIZE = 300
D_POS = 64
DG = 128
T = 100
B = 1024
ROWS = B * T
BBOX_DIM = 4 * D_POS
OUT_DIM = 512
ZVOFF = 304

CHUNK_R = 128
NW = 32
R_PER_W = ROWS // NW
NCHUNK = R_PER_W // CHUNK_R
UNITS = NW * NCHUNK


def _sc_gather_body(pk_hbm, table_hbm, poe_hbm, pk_v, idx_v, rows_v, sem):
    wid = lax.axis_index("s") * 2 + lax.axis_index("c")
    r0 = wid * R_PER_W

    def chunk(ci, carry):
        off = r0 + ci * CHUNK_R
        pltpu.sync_copy(pk_hbm.at[:, pl.ds(off, CHUNK_R)], pk_v)
        for c in range(4):
            for v in range(CHUNK_R // 16):
                f = lax.bitcast_convert_type(
                    pk_v[c, pl.ds(v * 16, 16)], jnp.float32
                )
                o = pk_v[4, pl.ds(v * 16, 16)]
                xi = (f * float(IMAGE_SIZE)).astype(jnp.int32)
                xi = jnp.minimum(jnp.maximum(xi, 0), IMAGE_SIZE)
                xi = jnp.where(o == 1, xi, xi + ZVOFF)
                idx_v[c, pl.ds(v * 16, 16)] = xi
        descs = [
            pltpu.async_copy(table_hbm.at[idx_v.at[c]], rows_v.at[c], sem)
            for c in range(4)
        ]
        for d in descs:
            d.wait()
        pltpu.sync_copy(rows_v, poe_hbm.at[wid * NCHUNK + ci])
        return carry

    lax.fori_loop(0, NCHUNK, chunk, 0)


def _sc_gather(packed, table):
    mesh = plsc.VectorSubcoreMesh(core_axis_name="c", subcore_axis_name="s")
    return functools.partial(
        pl.kernel,
        mesh=mesh,
        compiler_params=pltpu.CompilerParams(use_tc_tiling_on_sc=False),
        out_type=jax.ShapeDtypeStruct((UNITS, 4, CHUNK_R, DG), jnp.float32),
        scratch_types=[
            pltpu.VMEM((5, CHUNK_R), jnp.int32),
            pltpu.VMEM((4, CHUNK_R), jnp.int32),
            pltpu.VMEM((4, CHUNK_R, DG), jnp.float32),
            pltpu.SemaphoreType.DMA,
        ],
    )(_sc_gather_body)(packed, table)


def _base_body(wt_ref, ww_ref, b_ref, out_ref):
    acc = (
        jnp.dot(wt_ref[...], ww_ref[...], preferred_element_type=jnp.float32)
        + b_ref[...]
    )
    for k in range(R_PER_W // T):
        out_ref[pl.ds(k * T, T), :] = acc


def _mm_body(poe_ref, wq_ref, base_ref, out_ref):
    acc = base_ref[0]
    for c in range(4):
        acc = acc + jnp.dot(
            poe_ref[0, c], wq_ref[c], preferred_element_type=jnp.float32
        )
    out_ref[...] = jnp.maximum(acc, 0.0)


def kernel(batch_fractional_bboxs, batch_obj_vecs, pos_table, word_table, W, b):
    frac_t = batch_fractional_bboxs.reshape(ROWS, 4).T
    packed = jnp.concatenate(
        [lax.bitcast_convert_type(frac_t, jnp.int32),
         batch_obj_vecs.reshape(1, ROWS)], axis=0)
    table = jnp.pad(pos_table, ((0, 3 + ZVOFF), (0, DG - D_POS)))
    Wq = jnp.pad(
        W[:BBOX_DIM].reshape(4, D_POS, OUT_DIM),
        ((0, 0), (0, DG - D_POS), (0, 0)),
    )
    Ww = W[BBOX_DIM:]

    base_per = pl.pallas_call(
        _base_body,
        out_shape=jax.ShapeDtypeStruct((R_PER_W, OUT_DIM), jnp.float32),
    )(word_table, Ww, b.reshape(1, OUT_DIM))
    base25 = base_per.reshape(NCHUNK, CHUNK_R, OUT_DIM)

    poe = _sc_gather(packed, table)

    return pl.pallas_call(
        _mm_body,
        grid=(NCHUNK, NW),
        in_specs=[
            pl.BlockSpec((1, 4, CHUNK_R, DG),
                         lambda ci, w: (w * NCHUNK + ci, 0, 0, 0)),
            pl.BlockSpec((4, DG, OUT_DIM), lambda ci, w: (0, 0, 0)),
            pl.BlockSpec((1, CHUNK_R, OUT_DIM), lambda ci, w: (ci, 0, 0)),
        ],
        out_specs=pl.BlockSpec((CHUNK_R, OUT_DIM),
                               lambda ci, w: (w * NCHUNK + ci, 0)),
        out_shape=jax.ShapeDtypeStruct((ROWS, OUT_DIM), jnp.float32),
    )(poe, Wq, base25)


# speedup vs baseline: 11.7174x; 1.1695x over previous
import functools

import jax
import jax.numpy as jnp
from jax import lax
from jax.experimental import pallas as pl
from jax.experimental.pallas import tpu as pltpu
from jax.experimental.pallas import tpu_sc as plsc

IMAGE_SIZE = 300
D_POS = 64
DG = 128
T = 100
B = 1024
ROWS = B * T
BBOX_DIM = 4 * D_POS
OUT_DIM = 512
ZVOFF = 304

CHUNK_R = 128
GBLK = 128
NW = 32
R_PER_W = ROWS // NW
NCHUNK = R_PER_W // CHUNK_R
UNITS = NW * NCHUNK


def _sc_gather_body(pk_hbm, table_hbm, poe_hbm, pk_v, idx_v, rows_v, sem):
    wid = lax.axis_index("s") * 2 + lax.axis_index("c")
    r0 = wid * R_PER_W

    def chunk(ci, carry):
        off = r0 + ci * CHUNK_R
        pltpu.sync_copy(pk_hbm.at[:, pl.ds(off, CHUNK_R)], pk_v)
        for c in range(4):
            for v in range(CHUNK_R // 16):
                f = lax.bitcast_convert_type(
                    pk_v[c, pl.ds(v * 16, 16)], jnp.float32
                )
                o = pk_v[4, pl.ds(v * 16, 16)]
                xi = (f * float(IMAGE_SIZE)).astype(jnp.int32)
                xi = jnp.minimum(jnp.maximum(xi, 0), IMAGE_SIZE)
                xi = jnp.where(o == 1, xi, xi + ZVOFF)
                idx_v[c, pl.ds(v * 16, 16)] = xi
        descs = [
            pltpu.async_copy(table_hbm.at[idx_v.at[c]], rows_v.at[c], sem)
            for c in range(4)
        ]
        for d in descs:
            d.wait()
        u = wid * NCHUNK + ci
        for c in range(4):
            pltpu.sync_copy(
                rows_v.at[c],
                poe_hbm.at[u, c // 2, :, pl.ds(D_POS * (c % 2), D_POS)],
            )
        return carry

    lax.fori_loop(0, NCHUNK, chunk, 0)


def _sc_gather(packed, table):
    mesh = plsc.VectorSubcoreMesh(core_axis_name="c", subcore_axis_name="s")
    return functools.partial(
        pl.kernel,
        mesh=mesh,
        compiler_params=pltpu.CompilerParams(use_tc_tiling_on_sc=False),
        out_type=jax.ShapeDtypeStruct(
            (UNITS, 2, CHUNK_R, DG), jnp.float32),
        scratch_types=[
            pltpu.VMEM((5, CHUNK_R), jnp.int32),
            pltpu.VMEM((4, GBLK), jnp.int32),
            pltpu.VMEM((4, GBLK, D_POS), jnp.float32),
            pltpu.SemaphoreType.DMA,
        ],
    )(_sc_gather_body)(packed, table)


def _base_body(wt_ref, ww_ref, b_ref, out_ref):
    acc = (
        jnp.dot(wt_ref[...], ww_ref[...], preferred_element_type=jnp.float32)
        + b_ref[...]
    )
    for k in range(R_PER_W // T):
        out_ref[pl.ds(k * T, T), :] = acc


def _mm_body(poe_ref, wq_ref, base_ref, out_ref):
    acc = base_ref[0]
    for p in range(2):
        acc = acc + jnp.dot(
            poe_ref[0, p], wq_ref[p], preferred_element_type=jnp.float32
        )
    out_ref[...] = jnp.maximum(acc, 0.0)


def kernel(batch_fractional_bboxs, batch_obj_vecs, pos_table, word_table, W, b):
    frac_t = batch_fractional_bboxs.reshape(ROWS, 4).T
    packed = jnp.concatenate(
        [lax.bitcast_convert_type(frac_t, jnp.int32),
         batch_obj_vecs.reshape(1, ROWS)], axis=0)
    table = jnp.pad(pos_table, ((0, 3 + ZVOFF), (0, 0)))
    Wq = W[:BBOX_DIM].reshape(2, DG, OUT_DIM)
    Ww = W[BBOX_DIM:]

    base_per = pl.pallas_call(
        _base_body,
        out_shape=jax.ShapeDtypeStruct((R_PER_W, OUT_DIM), jnp.float32),
    )(word_table, Ww, b.reshape(1, OUT_DIM))
    base25 = base_per.reshape(NCHUNK, CHUNK_R, OUT_DIM)

    poe_r = _sc_gather(packed, table)

    return pl.pallas_call(
        _mm_body,
        grid=(NCHUNK, NW),
        in_specs=[
            pl.BlockSpec((1, 2, CHUNK_R, DG),
                         lambda ci, w: (w * NCHUNK + ci, 0, 0, 0)),
            pl.BlockSpec((2, DG, OUT_DIM), lambda ci, w: (0, 0, 0)),
            pl.BlockSpec((1, CHUNK_R, OUT_DIM), lambda ci, w: (ci, 0, 0)),
        ],
        out_specs=pl.BlockSpec((CHUNK_R, OUT_DIM),
                               lambda ci, w: (w * NCHUNK + ci, 0)),
        out_shape=jax.ShapeDtypeStruct((ROWS, OUT_DIM), jnp.float32),
    )(poe_r, Wq, base25)


# speedup vs baseline: 25.7667x; 2.1990x over previous
import functools

import jax
import jax.numpy as jnp
from jax import lax
from jax.experimental import pallas as pl
from jax.experimental.pallas import tpu as pltpu
from jax.experimental.pallas import tpu_sc as plsc

IMAGE_SIZE = 300
D_POS = 64
DG = 128
T = 100
B = 1024
ROWS = B * T
BBOX_DIM = 4 * D_POS
OUT_DIM = 512
ZVOFF = 304

CHUNK_R = 128
GBLK = 128
NW = 32
R_PER_W = ROWS // NW
NCHUNK = R_PER_W // CHUNK_R
UNITS = NW * NCHUNK
MB = 1600


def _sc_gather_body(pk_hbm, table_hbm, poe_hbm, pk_v, idx_v, rows_v, sem):
    wid = lax.axis_index("s") * 2 + lax.axis_index("c")
    r0 = wid * R_PER_W

    def chunk(ci, carry):
        off = r0 + ci * CHUNK_R
        pltpu.sync_copy(pk_hbm.at[:, pl.ds(off, CHUNK_R)], pk_v)
        for c in range(4):
            for v in range(CHUNK_R // 16):
                f = lax.bitcast_convert_type(
                    pk_v[c, pl.ds(v * 16, 16)], jnp.float32
                )
                o = pk_v[4, pl.ds(v * 16, 16)]
                xi = (f * float(IMAGE_SIZE)).astype(jnp.int32)
                xi = jnp.minimum(jnp.maximum(xi, 0), IMAGE_SIZE)
                xi = jnp.where(o == 1, xi, xi + ZVOFF)
                idx_v[c, pl.ds(v * 16, 16)] = xi
        descs = [
            pltpu.async_copy(table_hbm.at[idx_v.at[c]], rows_v.at[c], sem)
            for c in range(4)
        ]
        for d in descs:
            d.wait()
        for c in range(4):
            pltpu.sync_copy(
                rows_v.at[c],
                poe_hbm.at[wid, c // 2, pl.ds(ci * CHUNK_R, CHUNK_R),
                           pl.ds(D_POS * (c % 2), D_POS)],
            )
        return carry

    lax.fori_loop(0, NCHUNK, chunk, 0)


def _sc_gather(packed, table):
    mesh = plsc.VectorSubcoreMesh(core_axis_name="c", subcore_axis_name="s")
    return functools.partial(
        pl.kernel,
        mesh=mesh,
        compiler_params=pltpu.CompilerParams(use_tc_tiling_on_sc=False),
        out_type=jax.ShapeDtypeStruct(
            (NW, 2, R_PER_W, DG), jnp.float32),
        scratch_types=[
            pltpu.VMEM((5, CHUNK_R), jnp.int32),
            pltpu.VMEM((4, GBLK), jnp.int32),
            pltpu.VMEM((4, GBLK, D_POS), jnp.float32),
            pltpu.SemaphoreType.DMA,
        ],
    )(_sc_gather_body)(packed, table)


def _base_body(wt_ref, ww_ref, b_ref, out_ref):
    acc = (
        jnp.dot(wt_ref[...], ww_ref[...], preferred_element_type=jnp.float32)
        + b_ref[...]
    )
    for k in range(MB // T):
        out_ref[pl.ds(k * T, T), :] = acc


def _mm_body(poe_ref, wq_ref, base_ref, out_ref):
    acc = base_ref[...]
    for p in range(2):
        acc = acc + jnp.dot(
            poe_ref[0, p, 0], wq_ref[p], preferred_element_type=jnp.float32
        )
    out_ref[...] = jnp.maximum(acc, 0.0)


def kernel(batch_fractional_bboxs, batch_obj_vecs, pos_table, word_table, W, b):
    frac_t = batch_fractional_bboxs.reshape(ROWS, 4).T
    packed = jnp.concatenate(
        [lax.bitcast_convert_type(frac_t, jnp.int32),
         batch_obj_vecs.reshape(1, ROWS)], axis=0)
    table = jnp.pad(pos_table, ((0, 3 + ZVOFF), (0, 0)))
    Wq = W[:BBOX_DIM].reshape(2, DG, OUT_DIM)
    Ww = W[BBOX_DIM:]

    base_rep = pl.pallas_call(
        _base_body,
        out_shape=jax.ShapeDtypeStruct((MB, OUT_DIM), jnp.float32),
    )(word_table, Ww, b.reshape(1, OUT_DIM))

    poe_r = _sc_gather(packed, table)

    nh = R_PER_W // MB
    return pl.pallas_call(
        _mm_body,
        grid=(NW, nh),
        in_specs=[
            pl.BlockSpec((1, 2, MB, DG), lambda w, h: (w, 0, h, 0)),
            pl.BlockSpec((2, DG, OUT_DIM), lambda w, h: (0, 0, 0)),
            pl.BlockSpec((MB, OUT_DIM), lambda w, h: (0, 0)),
        ],
        out_specs=pl.BlockSpec((MB, OUT_DIM),
                               lambda w, h: (w * nh + h, 0)),
        out_shape=jax.ShapeDtypeStruct((ROWS, OUT_DIM), jnp.float32),
    )(poe_r, Wq, base_rep)
